# baseline (speedup 1.0000x reference)
import functools
import jax
import jax.numpy as jnp
from jax import lax
from jax.experimental import pallas as pl
from jax.experimental.pallas import tpu as pltpu
from jax.experimental.pallas import tpu_sc as plsc

_B, _S, _D = 4, 8192, 1024
_NW = 32
_PB = 2
_SPW = _S // (_NW // 2)
_CH = 16
_NCH = _SPW // _CH
_LANES = 16


def _sc_body(x_hbm, pos_hbm, out_hbm, pbuf, xbuf, insem, outsem):
    cid = lax.axis_index("c")
    sid = lax.axis_index("s")
    wid = sid * 2 + cid
    b0 = (wid // 16) * _PB
    s_base = (wid % 16) * _SPW

    def in_cps(i, slot):
        s0 = s_base + i * _CH
        return (
            pltpu.make_async_copy(
                pos_hbm.at[pl.ds(s0, _CH), :], pbuf.at[slot], insem.at[slot]),
            pltpu.make_async_copy(
                x_hbm.at[pl.ds(b0, _PB), pl.ds(s0, _CH), :], xbuf.at[slot],
                insem.at[slot]),
        )

    def out_cp(i, slot):
        s0 = s_base + i * _CH
        return pltpu.make_async_copy(
            xbuf.at[slot], out_hbm.at[pl.ds(b0, _PB), pl.ds(s0, _CH), :],
            outsem.at[slot])

    def compute(slot):
        @plsc.parallel_loop(0, _D // _LANES, unroll=2)
        def _(g):
            c = g * _LANES
            for r in range(_CH):
                pv = pbuf[slot, r, pl.ds(c, _LANES)]
                for b in range(_PB):
                    xbuf[slot, b, r, pl.ds(c, _LANES)] = (
                        xbuf[slot, b, r, pl.ds(c, _LANES)] + pv)

    def phase(i, slot):
        @pl.when(i + 1 < _NCH)
        def _():
            other = 1 - slot

            @pl.when(i >= 1)
            def _():
                out_cp(i - 1, other).wait()

            for d in in_cps(i + 1, other):
                d.start()

        for d in in_cps(i, slot):
            d.wait()
        compute(slot)
        out_cp(i, slot).start()

    for d in in_cps(0, 0):
        d.start()

    def kloop(k, carry):
        phase(k * 2, 0)
        phase(k * 2 + 1, 1)
        return carry

    lax.fori_loop(0, _NCH // 2, kloop, 0)
    out_cp(_NCH - 2, 0).wait()
    out_cp(_NCH - 1, 1).wait()


_sc_kernel = functools.partial(
    pl.kernel,
    out_type=jax.ShapeDtypeStruct((_B, _S, _D), jnp.float32),
    mesh=plsc.VectorSubcoreMesh(core_axis_name="c", subcore_axis_name="s"),
    scratch_types=[
        pltpu.VMEM((2, _CH, _D), jnp.float32),
        pltpu.VMEM((2, _PB, _CH, _D), jnp.float32),
        pltpu.SemaphoreType.DMA((2,)),
        pltpu.SemaphoreType.DMA((2,)),
    ],
    compiler_params=pltpu.CompilerParams(use_tc_tiling_on_sc=True),
)(_sc_body)


def kernel(x, pos_table):
    B, S, D = x.shape
    return _sc_kernel(x, pos_table[:S])


# speedup vs baseline: 1.1202x; 1.1202x over previous
import functools
import jax
import jax.numpy as jnp
from jax import lax
from jax.experimental import pallas as pl
from jax.experimental.pallas import tpu as pltpu
from jax.experimental.pallas import tpu_sc as plsc

_B, _S, _D = 4, 8192, 1024
_NW = 32
_SPW = _S // _NW
_CH = 8
_NCH = _SPW // _CH
_LANES = 16


def _sc_body(x_hbm, pos_hbm, out_hbm, pbuf, xbuf, insem, outsem):
    cid = lax.axis_index("c")
    sid = lax.axis_index("s")
    wid = sid * 2 + cid
    s_base = wid * _SPW

    def in_cps(i, slot):
        s0 = s_base + i * _CH
        return (
            pltpu.make_async_copy(
                pos_hbm.at[pl.ds(s0, _CH), :], pbuf.at[slot], insem.at[slot]),
            pltpu.make_async_copy(
                x_hbm.at[:, pl.ds(s0, _CH), :], xbuf.at[slot], insem.at[slot]),
        )

    def out_cp(i, slot):
        s0 = s_base + i * _CH
        return pltpu.make_async_copy(
            xbuf.at[slot], out_hbm.at[:, pl.ds(s0, _CH), :], outsem.at[slot])

    def compute(slot):
        @plsc.parallel_loop(0, _D // _LANES, unroll=2)
        def _(g):
            c = g * _LANES
            for r in range(_CH):
                pv = pbuf[slot, r, pl.ds(c, _LANES)]
                for b in range(_B):
                    xbuf[slot, b, r, pl.ds(c, _LANES)] = (
                        xbuf[slot, b, r, pl.ds(c, _LANES)] + pv)

    def phase(i, slot):
        @pl.when(i + 1 < _NCH)
        def _():
            other = 1 - slot

            @pl.when(i >= 1)
            def _():
                out_cp(i - 1, other).wait()

            for d in in_cps(i + 1, other):
                d.start()

        for d in in_cps(i, slot):
            d.wait()
        compute(slot)
        out_cp(i, slot).start()

    for d in in_cps(0, 0):
        d.start()

    def kloop(k, carry):
        phase(k * 2, 0)
        phase(k * 2 + 1, 1)
        return carry

    lax.fori_loop(0, _NCH // 2, kloop, 0)
    out_cp(_NCH - 2, 0).wait()
    out_cp(_NCH - 1, 1).wait()


_sc_kernel = functools.partial(
    pl.kernel,
    out_type=jax.ShapeDtypeStruct((_B, _S, _D), jnp.float32),
    mesh=plsc.VectorSubcoreMesh(core_axis_name="c", subcore_axis_name="s"),
    scratch_types=[
        pltpu.VMEM((2, _CH, _D), jnp.float32),
        pltpu.VMEM((2, _B, _CH, _D), jnp.float32),
        pltpu.SemaphoreType.DMA((2,)),
        pltpu.SemaphoreType.DMA((2,)),
    ],
    compiler_params=pltpu.CompilerParams(use_tc_tiling_on_sc=True),
)(_sc_body)


def kernel(x, pos_table):
    B, S, D = x.shape
    return _sc_kernel(x, pos_table[:S])
